# baseline (speedup 1.0000x reference)
import functools

import jax
import jax.numpy as jnp
from jax import lax
from jax.experimental import pallas as pl
from jax.experimental.pallas import tpu as pltpu
from jax.experimental.pallas import tpu_sc as plsc

N = 10000
E = 320000
D = 128
H = 256
B = 64

NC = 2
NS = 16
NW = NC * NS

PIECES = 2
EP = E // PIECES
CHUNK = 128
PCHUNKS = EP // CHUNK
PCH_PAD = 1280
PERW = PCH_PAD // NW
GRP = 1
NGRP = PERW // GRP
GTOT = PCH_PAD // GRP
GPAD = GTOT + 16
PREFG = NGRP + 16
GG = GRP * CHUNK
NODE_STRIPE = 624
TAIL_BASE = NODE_STRIPE * NS
NPAD = N

_mesh = functools.partial(
    plsc.VectorSubcoreMesh, core_axis_name="c", subcore_axis_name="s",
    num_cores=NC, num_subcores=NS)


def _gather_body(x_hbm, row2_hbm, col2_hbm, src_hbm, dst_hbm,
                 idxp, rows0, rows1, sg0, sg1, so0, so1):
    cid = lax.axis_index("c")
    sid = lax.axis_index("s")
    wid = sid * NC + cid
    gstart = NGRP * wid
    astart = (gstart // 8) * 8
    off = gstart - astart

    pltpu.sync_copy(row2_hbm.at[pl.ds(astart, PREFG)], idxp.at[0])
    pltpu.sync_copy(col2_hbm.at[pl.ds(astart, PREFG)], idxp.at[1])

    bufs = (rows0, rows1)
    sg = (sg0, sg1)
    so = (so0, so1)
    outs = (src_hbm, dst_hbm)

    tasks = [(u % 2, u // 2) for u in range(2 * NGRP)]

    def idx_ref(kind, g):
        return idxp.at[kind, off + g]

    def out_copy(kind, g, b):
        base = (gstart + g) * GG
        return pltpu.async_copy(
            bufs[b], outs[kind].at[pl.ds(base, GG)], so[b])

    for u, (kind, g) in enumerate(tasks):
        b = u % 2
        if u >= 2:
            pltpu.make_async_copy(
                bufs[b], outs[0].at[pl.ds(0, GG)], so[b]).wait()
        pltpu.async_copy(x_hbm.at[idx_ref(kind, g)], bufs[b], sg[b])
        if u >= 1:
            pk, pg = tasks[u - 1]
            pb = (u - 1) % 2
            pltpu.make_async_copy(
                x_hbm.at[idx_ref(pk, pg)], bufs[pb], sg[pb]).wait()
            out_copy(pk, pg, pb)
    lk, lg = tasks[-1]
    lb = (2 * NGRP - 1) % 2
    pltpu.make_async_copy(
        x_hbm.at[idx_ref(lk, lg)], bufs[lb], sg[lb]).wait()
    out_copy(lk, lg, lb)
    for b in (0, 1):
        pltpu.make_async_copy(
            bufs[b], outs[0].at[pl.ds(0, GG)], so[b]).wait()


def _sc_gather(xp, row2, col2):
    return pl.kernel(
        _gather_body,
        out_type=(jax.ShapeDtypeStruct((PCH_PAD * CHUNK, D), jnp.float32),
                  jax.ShapeDtypeStruct((PCH_PAD * CHUNK, D), jnp.float32)),
        mesh=_mesh(),
        scratch_types=[
            pltpu.VMEM((2, PREFG, GG), jnp.int32),
            pltpu.VMEM((GG, D), jnp.float32),
            pltpu.VMEM((GG, D), jnp.float32),
            pltpu.SemaphoreType.DMA,
            pltpu.SemaphoreType.DMA,
            pltpu.SemaphoreType.DMA,
            pltpu.SemaphoreType.DMA,
        ],
    )(xp, row2, col2)


def _scatter_body(e_hbm, col2_hbm, zeros_hbm, agg_hbm,
                  idxp, rows0, rows1, acc_sp, sl0, sl1, sa0, sa1):
    cid = lax.axis_index("c")
    sid = lax.axis_index("s")
    wid = sid * NC + cid
    gstart = NGRP * wid
    astart = (gstart // 8) * 8
    off = gstart - astart

    stripe = sid * NODE_STRIPE
    pltpu.sync_copy(zeros_hbm.at[pl.ds(stripe, NODE_STRIPE)],
                    acc_sp.at[pl.ds(stripe, NODE_STRIPE)])
    tail = TAIL_BASE + sid * 8

    @pl.when(sid < (N - TAIL_BASE) // 8)
    def _():
        pltpu.sync_copy(zeros_hbm.at[pl.ds(tail, 8)], acc_sp.at[pl.ds(tail, 8)])

    pltpu.sync_copy(col2_hbm.at[pl.ds(astart, PREFG)], idxp)
    plsc.subcore_barrier()

    bufs = (rows0, rows1)
    sl = (sl0, sl1)
    sa = (sa0, sa1)

    def load(g, b):
        base = jnp.minimum((gstart + g) * GG, EP - GG)
        return pltpu.async_copy(
            e_hbm.at[pl.ds(base, GG)], bufs[b], sl[b])

    def add(g, b):
        return pltpu.async_copy(
            bufs[b], acc_sp.at[idxp.at[off + g]], sa[b], add=True)

    for g in range(NGRP):
        b = g % 2
        if g >= 2:
            pltpu.make_async_copy(
                bufs[b], acc_sp.at[idxp.at[off]], sa[b]).wait()
        load(g, b)
        if g >= 1:
            pb = (g - 1) % 2
            pltpu.make_async_copy(
                e_hbm.at[pl.ds(0, GG)], bufs[pb], sl[pb]).wait()
            add(g - 1, pb)
    lb = (NGRP - 1) % 2
    pltpu.make_async_copy(
        e_hbm.at[pl.ds(0, GG)], bufs[lb], sl[lb]).wait()
    add(NGRP - 1, lb)
    for b in (0, 1):
        pltpu.make_async_copy(
            bufs[b], acc_sp.at[idxp.at[off]], sa[b]).wait()

    plsc.subcore_barrier()
    pltpu.sync_copy(acc_sp.at[pl.ds(stripe, NODE_STRIPE)],
                    agg_hbm.at[cid, pl.ds(stripe, NODE_STRIPE)])

    @pl.when(sid < (N - TAIL_BASE) // 8)
    def _():
        pltpu.sync_copy(acc_sp.at[pl.ds(tail, 8)],
                        agg_hbm.at[cid, pl.ds(tail, 8)])


def _sc_scatter(e_new, col2, zeros):
    return pl.kernel(
        _scatter_body,
        out_type=jax.ShapeDtypeStruct((NC, N, D), jnp.float32),
        mesh=_mesh(),
        scratch_types=[
            pltpu.VMEM((PREFG, GG), jnp.int32),
            pltpu.VMEM((GG, D), jnp.float32),
            pltpu.VMEM((GG, D), jnp.float32),
            pltpu.VMEM_SHARED((N + 8, D), jnp.float32),
            pltpu.SemaphoreType.DMA,
            pltpu.SemaphoreType.DMA,
            pltpu.SemaphoreType.DMA,
            pltpu.SemaphoreType.DMA,
        ],
    )(e_new, col2, zeros)


EB = 1600


def _edge_mlp_math(src, dst, ea, w1a, w1b, w1c, b1, w2, b2):
    h = jnp.dot(src[...], w1a[...], preferred_element_type=jnp.float32)
    h += jnp.dot(dst[...], w1b[...], preferred_element_type=jnp.float32)
    h += jnp.dot(ea[...], w1c[...], preferred_element_type=jnp.float32)
    h = jnp.maximum(h + b1[...], 0.0)
    return jnp.dot(h, w2[...], preferred_element_type=jnp.float32) + b2[...]


def _edge_mlp_body0(src, dst, ea, w1a, w1b, w1c, b1, w2, b2, out, full_out):
    e = _edge_mlp_math(src, dst, ea, w1a, w1b, w1c, b1, w2, b2)
    out[...] = e
    full_out[...] = e


def _edge_mlp_bodyk(src, dst, ea, w1a, w1b, w1c, b1, w2, b2, buf,
                    out, full_out):
    del buf
    e = _edge_mlp_math(src, dst, ea, w1a, w1b, w1c, b1, w2, b2)
    out[...] = e
    full_out[...] = e


def _tc_edge_mlp(k, src, dst, ea, w1a, w1b, w1c, b1, w2, b2, e_buf):
    rows = lambda i: (i, 0)
    off = k * (EP // EB)
    rows_off = lambda i: (i + off, 0)
    full = lambda i: (0, 0)
    in_specs = [
        pl.BlockSpec((EB, D), rows),
        pl.BlockSpec((EB, D), rows),
        pl.BlockSpec((EB, D), rows),
        pl.BlockSpec((D, H), full),
        pl.BlockSpec((D, H), full),
        pl.BlockSpec((D, H), full),
        pl.BlockSpec((1, H), full),
        pl.BlockSpec((H, D), full),
        pl.BlockSpec((1, D), full),
    ]
    args = (src, dst, ea, w1a, w1b, w1c, b1, w2, b2)
    if k == 0:
        body = _edge_mlp_body0
        aliases = {}
    else:
        body = _edge_mlp_bodyk
        in_specs.append(pl.BlockSpec(memory_space=pl.ANY))
        args = args + (e_buf,)
        aliases = {9: 1}
    return pl.pallas_call(
        body,
        grid=(EP // EB,),
        in_specs=in_specs,
        out_specs=[pl.BlockSpec((EB, D), rows),
                   pl.BlockSpec((EB, D), rows_off)],
        out_shape=[jax.ShapeDtypeStruct((EP, D), jnp.float32),
                   jax.ShapeDtypeStruct((E, D), jnp.float32)],
        input_output_aliases=aliases,
    )(*args)


NB = 1000
NGRID = N // NB


def _node_global_body(x, a0, a1, e1, batch, wn1a, wn1b, bn1, wn2, bn2,
                      wg1a, wg1b, bg1, wg2, bg2,
                      x_out, u_out, nsum, esum, cnt):
    i = pl.program_id(0)

    @pl.when(i == 0)
    def _():
        nsum[...] = jnp.zeros_like(nsum)
        esum[...] = jnp.zeros_like(esum)
        cnt[...] = jnp.zeros_like(cnt)

    a = a0[0] + a0[1] + a1[0] + a1[1]
    h = jnp.dot(x[...], wn1a[...], preferred_element_type=jnp.float32)
    h += jnp.dot(a, wn1b[...], preferred_element_type=jnp.float32)
    h = jnp.maximum(h + bn1[...], 0.0)
    xn = jnp.dot(h, wn2[...], preferred_element_type=jnp.float32) + bn2[...]
    x_out[...] = xn

    gid = lax.broadcasted_iota(jnp.int32, (B, NB), 0)
    oh = (gid == batch[0]).astype(jnp.float32)
    nsum[...] += jnp.dot(oh, xn, preferred_element_type=jnp.float32)
    esum[...] += jnp.dot(oh, e1[...], preferred_element_type=jnp.float32)
    cnt[...] += jnp.broadcast_to(jnp.sum(oh, axis=1, keepdims=True), (B, D))

    @pl.when(i == NGRID - 1)
    def _():
        denom = cnt[...] + 1e-6
        nmean = nsum[...] / denom
        emean = esum[...] / denom
        hg = jnp.dot(nmean, wg1a[...], preferred_element_type=jnp.float32)
        hg += jnp.dot(emean, wg1b[...], preferred_element_type=jnp.float32)
        hg = jnp.maximum(hg + bg1[...], 0.0)
        u_out[...] = jnp.dot(hg, wg2[...],
                             preferred_element_type=jnp.float32) + bg2[...]


def _tc_node_global(x, aggs, e_new0, batch3, wn1a, wn1b, bn1, wn2, bn2,
                    wg1a, wg1b, bg1, wg2, bg2):
    rows = lambda i: (i, 0)
    full = lambda i: (0, 0)
    agg_spec = pl.BlockSpec((NC, NB, D), lambda i: (0, i, 0))
    return pl.pallas_call(
        _node_global_body,
        grid=(NGRID,),
        in_specs=[
            pl.BlockSpec((NB, D), rows),
            agg_spec, agg_spec,
            pl.BlockSpec((NB, D), rows),
            pl.BlockSpec((1, 1, NB), lambda i: (i, 0, 0)),
            pl.BlockSpec((D, H), full),
            pl.BlockSpec((D, H), full),
            pl.BlockSpec((1, H), full),
            pl.BlockSpec((H, D), full),
            pl.BlockSpec((1, D), full),
            pl.BlockSpec((D, H), full),
            pl.BlockSpec((D, H), full),
            pl.BlockSpec((1, H), full),
            pl.BlockSpec((H, D), full),
            pl.BlockSpec((1, D), full),
        ],
        out_specs=[
            pl.BlockSpec((NB, D), rows),
            pl.BlockSpec((B, D), full),
        ],
        out_shape=[
            jax.ShapeDtypeStruct((N, D), jnp.float32),
            jax.ShapeDtypeStruct((B, D), jnp.float32),
        ],
        scratch_shapes=[
            pltpu.VMEM((B, D), jnp.float32),
            pltpu.VMEM((B, D), jnp.float32),
            pltpu.VMEM((B, D), jnp.float32),
        ],
    )(x, *aggs, e_new0, batch3, wn1a, wn1b, bn1, wn2, bn2,
      wg1a, wg1b, bg1, wg2, bg2)


def kernel(x, edge_index, edge_attr, u, batch,
           We1, be1, We2, be2,
           Wn1, bn1, Wn2, bn2,
           Wg1, bg1, Wg2, bg2):
    row = edge_index[0]
    col = edge_index[1]
    zeros = jnp.zeros((N, D), jnp.float32)
    xp = jnp.pad(x, ((0, 8), (0, 0)))
    pad = (0, GPAD * GG - EP)

    w1a, w1b, w1c = We1[:D], We1[D:2 * D], We1[2 * D:]
    b1 = be1.reshape(1, H)
    b2 = be2.reshape(1, D)

    e_pieces = []
    aggs = []
    e_new = None
    for k in range(PIECES):
        sl = slice(k * EP, (k + 1) * EP)
        row2 = jnp.pad(row[sl], pad,
                       constant_values=NPAD).reshape(GPAD, GG)
        col2 = jnp.pad(col[sl], pad,
                       constant_values=NPAD).reshape(GPAD, GG)
        srcg, dstg = _sc_gather(xp, row2, col2)
        e_k, e_new = _tc_edge_mlp(k, srcg, dstg, edge_attr[sl],
                                  w1a, w1b, w1c, b1, We2, b2, e_new)
        e_pieces.append(e_k)
        aggs.append(_sc_scatter(e_k, col2, zeros))

    batch3 = batch.reshape(NGRID, 1, NB)
    x_new, u_new = _tc_node_global(
        x, aggs, e_pieces[0], batch3,
        Wn1[:D], Wn1[D:], bn1.reshape(1, H), Wn2, bn2.reshape(1, D),
        Wg1[:D], Wg1[D:], bg1.reshape(1, H), Wg2, bg2.reshape(1, D))

    return x_new, e_new, u_new


# speedup vs baseline: 1.0071x; 1.0071x over previous
import functools

import jax
import jax.numpy as jnp
from jax import lax
from jax.experimental import pallas as pl
from jax.experimental.pallas import tpu as pltpu
from jax.experimental.pallas import tpu_sc as plsc

N = 10000
E = 320000
D = 128
H = 256
B = 64

NC = 2
NS = 16
NW = NC * NS

PIECES = 2
EP = E // PIECES
CHUNK = 128
PCHUNKS = EP // CHUNK
PCH_PAD = 1280
PERW = PCH_PAD // NW
GRP = 1
NGRP = PERW // GRP
GTOT = PCH_PAD // GRP
GPAD = GTOT + 16
PREFG = NGRP + 16
GG = GRP * CHUNK
NODE_STRIPE = 624
TAIL_BASE = NODE_STRIPE * NS
NPAD = N

_mesh = functools.partial(
    plsc.VectorSubcoreMesh, core_axis_name="c", subcore_axis_name="s",
    num_cores=NC, num_subcores=NS)


def _gather_body(x_hbm, row2_hbm, col2_hbm, src_hbm, dst_hbm,
                 idxp, rows0, rows1, rows2, rows3,
                 sg0, sg1, sg2, sg3, so0, so1, so2, so3):
    cid = lax.axis_index("c")
    sid = lax.axis_index("s")
    wid = sid * NC + cid
    gstart = NGRP * wid
    astart = (gstart // 8) * 8
    off = gstart - astart

    pltpu.sync_copy(row2_hbm.at[pl.ds(astart, PREFG)], idxp.at[0])
    pltpu.sync_copy(col2_hbm.at[pl.ds(astart, PREFG)], idxp.at[1])

    bufs = (rows0, rows1, rows2, rows3)
    sg = (sg0, sg1, sg2, sg3)
    so = (so0, so1, so2, so3)
    outs = (src_hbm, dst_hbm)

    T = 2 * NGRP

    def gather(u_g, b, kind):
        pltpu.async_copy(x_hbm.at[idxp.at[kind, off + u_g]], bufs[b], sg[b])

    def out_start(u_g, b, kind):
        pltpu.async_copy(
            bufs[b], outs[kind].at[pl.ds((gstart + u_g) * CHUNK, CHUNK)],
            so[b])

    def wait_gather(b):
        pltpu.make_async_copy(
            x_hbm.at[idxp.at[0, off]], bufs[b], sg[b]).wait()

    def wait_out(b):
        pltpu.make_async_copy(
            bufs[b], outs[0].at[pl.ds(0, CHUNK)], so[b]).wait()

    gather(0, 0, 0)
    gather(0, 1, 1)
    wait_gather(0); out_start(0, 0, 0)
    gather(1, 2, 0)
    wait_gather(1); out_start(0, 1, 1)
    gather(1, 3, 1)
    wait_gather(2); out_start(1, 2, 0)

    def step(j, carry):
        for b in range(4):
            kind = b % 2
            g = 2 * j + b // 2
            wait_out(b)
            gather(g, b, kind)
            pb = (b - 1) % 4
            pg = 2 * j + (b - 1) // 2 if b > 0 else 2 * (j - 1) + 1
            wait_gather(pb)
            out_start(pg, pb, pb % 2)
        return carry

    lax.fori_loop(1, T // 4, step, 0)
    wait_gather(3)
    out_start(2 * (T // 4 - 1) + 1, 3, 1)
    for b in range(4):
        wait_out(b)


def _sc_gather(xp, row2, col2):
    return pl.kernel(
        _gather_body,
        out_type=(jax.ShapeDtypeStruct((PCH_PAD * CHUNK, D), jnp.float32),
                  jax.ShapeDtypeStruct((PCH_PAD * CHUNK, D), jnp.float32)),
        mesh=_mesh(),
        scratch_types=(
            [pltpu.VMEM((2, PREFG, GG), jnp.int32)]
            + [pltpu.VMEM((GG, D), jnp.float32)] * 4
            + [pltpu.SemaphoreType.DMA] * 8
        ),
    )(xp, row2, col2)


def _scatter_body(e_hbm, col2_hbm, zeros_hbm, agg_hbm,
                  idxp, rows0, rows1, acc_sp, sl0, sl1, sa0, sa1):
    cid = lax.axis_index("c")
    sid = lax.axis_index("s")
    wid = sid * NC + cid
    gstart = NGRP * wid
    astart = (gstart // 8) * 8
    off = gstart - astart

    stripe = sid * NODE_STRIPE
    pltpu.sync_copy(zeros_hbm.at[pl.ds(stripe, NODE_STRIPE)],
                    acc_sp.at[pl.ds(stripe, NODE_STRIPE)])
    tail = TAIL_BASE + sid * 8

    @pl.when(sid < (N - TAIL_BASE) // 8)
    def _():
        pltpu.sync_copy(zeros_hbm.at[pl.ds(tail, 8)], acc_sp.at[pl.ds(tail, 8)])

    pltpu.sync_copy(col2_hbm.at[pl.ds(astart, PREFG)], idxp)
    plsc.subcore_barrier()

    bufs = (rows0, rows1)
    sl = (sl0, sl1)
    sa = (sa0, sa1)

    def load(g, b):
        base = jnp.minimum((gstart + g) * GG, EP - GG)
        return pltpu.async_copy(
            e_hbm.at[pl.ds(base, GG)], bufs[b], sl[b])

    def add(g, b):
        return pltpu.async_copy(
            bufs[b], acc_sp.at[idxp.at[off + g]], sa[b], add=True)

    for g in range(NGRP):
        b = g % 2
        if g >= 2:
            pltpu.make_async_copy(
                bufs[b], acc_sp.at[idxp.at[off]], sa[b]).wait()
        load(g, b)
        if g >= 1:
            pb = (g - 1) % 2
            pltpu.make_async_copy(
                e_hbm.at[pl.ds(0, GG)], bufs[pb], sl[pb]).wait()
            add(g - 1, pb)
    lb = (NGRP - 1) % 2
    pltpu.make_async_copy(
        e_hbm.at[pl.ds(0, GG)], bufs[lb], sl[lb]).wait()
    add(NGRP - 1, lb)
    for b in (0, 1):
        pltpu.make_async_copy(
            bufs[b], acc_sp.at[idxp.at[off]], sa[b]).wait()

    plsc.subcore_barrier()
    pltpu.sync_copy(acc_sp.at[pl.ds(stripe, NODE_STRIPE)],
                    agg_hbm.at[cid, pl.ds(stripe, NODE_STRIPE)])

    @pl.when(sid < (N - TAIL_BASE) // 8)
    def _():
        pltpu.sync_copy(acc_sp.at[pl.ds(tail, 8)],
                        agg_hbm.at[cid, pl.ds(tail, 8)])


def _sc_scatter(e_new, col2, zeros):
    return pl.kernel(
        _scatter_body,
        out_type=jax.ShapeDtypeStruct((NC, N, D), jnp.float32),
        mesh=_mesh(),
        scratch_types=[
            pltpu.VMEM((PREFG, GG), jnp.int32),
            pltpu.VMEM((GG, D), jnp.float32),
            pltpu.VMEM((GG, D), jnp.float32),
            pltpu.VMEM_SHARED((N + 8, D), jnp.float32),
            pltpu.SemaphoreType.DMA,
            pltpu.SemaphoreType.DMA,
            pltpu.SemaphoreType.DMA,
            pltpu.SemaphoreType.DMA,
        ],
    )(e_new, col2, zeros)


EB = 1600


def _edge_mlp_math(src, dst, ea, w1a, w1b, w1c, b1, w2, b2):
    h = jnp.dot(src[...], w1a[...], preferred_element_type=jnp.float32)
    h += jnp.dot(dst[...], w1b[...], preferred_element_type=jnp.float32)
    h += jnp.dot(ea[...], w1c[...], preferred_element_type=jnp.float32)
    h = jnp.maximum(h + b1[...], 0.0)
    return jnp.dot(h, w2[...], preferred_element_type=jnp.float32) + b2[...]


def _edge_mlp_body0(src, dst, ea, w1a, w1b, w1c, b1, w2, b2, out, full_out):
    e = _edge_mlp_math(src, dst, ea, w1a, w1b, w1c, b1, w2, b2)
    out[...] = e
    full_out[...] = e


def _edge_mlp_bodyk(src, dst, ea, w1a, w1b, w1c, b1, w2, b2, buf,
                    out, full_out):
    del buf
    e = _edge_mlp_math(src, dst, ea, w1a, w1b, w1c, b1, w2, b2)
    out[...] = e
    full_out[...] = e


def _tc_edge_mlp(k, src, dst, ea, w1a, w1b, w1c, b1, w2, b2, e_buf):
    rows = lambda i: (i, 0)
    off = k * (EP // EB)
    rows_off = lambda i: (i + off, 0)
    full = lambda i: (0, 0)
    in_specs = [
        pl.BlockSpec((EB, D), rows),
        pl.BlockSpec((EB, D), rows),
        pl.BlockSpec((EB, D), rows),
        pl.BlockSpec((D, H), full),
        pl.BlockSpec((D, H), full),
        pl.BlockSpec((D, H), full),
        pl.BlockSpec((1, H), full),
        pl.BlockSpec((H, D), full),
        pl.BlockSpec((1, D), full),
    ]
    args = (src, dst, ea, w1a, w1b, w1c, b1, w2, b2)
    if k == 0:
        body = _edge_mlp_body0
        aliases = {}
    else:
        body = _edge_mlp_bodyk
        in_specs.append(pl.BlockSpec(memory_space=pl.ANY))
        args = args + (e_buf,)
        aliases = {9: 1}
    return pl.pallas_call(
        body,
        grid=(EP // EB,),
        in_specs=in_specs,
        out_specs=[pl.BlockSpec((EB, D), rows),
                   pl.BlockSpec((EB, D), rows_off)],
        out_shape=[jax.ShapeDtypeStruct((EP, D), jnp.float32),
                   jax.ShapeDtypeStruct((E, D), jnp.float32)],
        input_output_aliases=aliases,
    )(*args)


NB = 1000
NGRID = N // NB


def _node_global_body(x, a0, a1, e1, batch, wn1a, wn1b, bn1, wn2, bn2,
                      wg1a, wg1b, bg1, wg2, bg2,
                      x_out, u_out, nsum, esum, cnt):
    i = pl.program_id(0)

    @pl.when(i == 0)
    def _():
        nsum[...] = jnp.zeros_like(nsum)
        esum[...] = jnp.zeros_like(esum)
        cnt[...] = jnp.zeros_like(cnt)

    a = a0[0] + a0[1] + a1[0] + a1[1]
    h = jnp.dot(x[...], wn1a[...], preferred_element_type=jnp.float32)
    h += jnp.dot(a, wn1b[...], preferred_element_type=jnp.float32)
    h = jnp.maximum(h + bn1[...], 0.0)
    xn = jnp.dot(h, wn2[...], preferred_element_type=jnp.float32) + bn2[...]
    x_out[...] = xn

    gid = lax.broadcasted_iota(jnp.int32, (B, NB), 0)
    oh = (gid == batch[0]).astype(jnp.float32)
    nsum[...] += jnp.dot(oh, xn, preferred_element_type=jnp.float32)
    esum[...] += jnp.dot(oh, e1[...], preferred_element_type=jnp.float32)
    cnt[...] += jnp.broadcast_to(jnp.sum(oh, axis=1, keepdims=True), (B, D))

    @pl.when(i == NGRID - 1)
    def _():
        denom = cnt[...] + 1e-6
        nmean = nsum[...] / denom
        emean = esum[...] / denom
        hg = jnp.dot(nmean, wg1a[...], preferred_element_type=jnp.float32)
        hg += jnp.dot(emean, wg1b[...], preferred_element_type=jnp.float32)
        hg = jnp.maximum(hg + bg1[...], 0.0)
        u_out[...] = jnp.dot(hg, wg2[...],
                             preferred_element_type=jnp.float32) + bg2[...]


def _tc_node_global(x, aggs, e_new0, batch3, wn1a, wn1b, bn1, wn2, bn2,
                    wg1a, wg1b, bg1, wg2, bg2):
    rows = lambda i: (i, 0)
    full = lambda i: (0, 0)
    agg_spec = pl.BlockSpec((NC, NB, D), lambda i: (0, i, 0))
    return pl.pallas_call(
        _node_global_body,
        grid=(NGRID,),
        in_specs=[
            pl.BlockSpec((NB, D), rows),
            agg_spec, agg_spec,
            pl.BlockSpec((NB, D), rows),
            pl.BlockSpec((1, 1, NB), lambda i: (i, 0, 0)),
            pl.BlockSpec((D, H), full),
            pl.BlockSpec((D, H), full),
            pl.BlockSpec((1, H), full),
            pl.BlockSpec((H, D), full),
            pl.BlockSpec((1, D), full),
            pl.BlockSpec((D, H), full),
            pl.BlockSpec((D, H), full),
            pl.BlockSpec((1, H), full),
            pl.BlockSpec((H, D), full),
            pl.BlockSpec((1, D), full),
        ],
        out_specs=[
            pl.BlockSpec((NB, D), rows),
            pl.BlockSpec((B, D), full),
        ],
        out_shape=[
            jax.ShapeDtypeStruct((N, D), jnp.float32),
            jax.ShapeDtypeStruct((B, D), jnp.float32),
        ],
        scratch_shapes=[
            pltpu.VMEM((B, D), jnp.float32),
            pltpu.VMEM((B, D), jnp.float32),
            pltpu.VMEM((B, D), jnp.float32),
        ],
    )(x, *aggs, e_new0, batch3, wn1a, wn1b, bn1, wn2, bn2,
      wg1a, wg1b, bg1, wg2, bg2)


def kernel(x, edge_index, edge_attr, u, batch,
           We1, be1, We2, be2,
           Wn1, bn1, Wn2, bn2,
           Wg1, bg1, Wg2, bg2):
    row = edge_index[0]
    col = edge_index[1]
    zeros = jnp.zeros((N, D), jnp.float32)
    xp = jnp.pad(x, ((0, 8), (0, 0)))
    pad = (0, GPAD * GG - EP)

    w1a, w1b, w1c = We1[:D], We1[D:2 * D], We1[2 * D:]
    b1 = be1.reshape(1, H)
    b2 = be2.reshape(1, D)

    e_pieces = []
    aggs = []
    e_new = None
    for k in range(PIECES):
        sl = slice(k * EP, (k + 1) * EP)
        row2 = jnp.pad(row[sl], pad,
                       constant_values=NPAD).reshape(GPAD, GG)
        col2 = jnp.pad(col[sl], pad,
                       constant_values=NPAD).reshape(GPAD, GG)
        srcg, dstg = _sc_gather(xp, row2, col2)
        e_k, e_new = _tc_edge_mlp(k, srcg, dstg, edge_attr[sl],
                                  w1a, w1b, w1c, b1, We2, b2, e_new)
        e_pieces.append(e_k)
        aggs.append(_sc_scatter(e_k, col2, zeros))

    batch3 = batch.reshape(NGRID, 1, NB)
    x_new, u_new = _tc_node_global(
        x, aggs, e_pieces[0], batch3,
        Wn1[:D], Wn1[D:], bn1.reshape(1, H), Wn2, bn2.reshape(1, D),
        Wg1[:D], Wg1[D:], bg1.reshape(1, H), Wg2, bg2.reshape(1, D))

    return x_new, e_new, u_new
